# baseline (speedup 1.0000x reference)
import functools

import jax
import jax.numpy as jnp
from jax import lax
from jax.experimental import pallas as pl
from jax.experimental.pallas import tpu as pltpu
from jax.experimental.pallas import tpu_sc as plsc

_N = 10000
_E = 320000
_H = 128
_ITER = 3

_NC = 2
_NS = 16
_NW = _NC * _NS
_C = 128
_PW = 80
_EP = _NW * _PW * _C
_NBUF = 2
_IB = 16
_IR = 3 * _IB
_NBLK = _PW // _IB
_NP = 10240
_RPS = _NP // _NS

_BN = 1000
_G = _N // _BN


_HC = _C // 2


def _issue_gather(x_hbm, src_i, rows, gsems, b, rrow):
    pltpu.async_copy(x_hbm.at[src_i.at[rrow, pl.ds(0, _HC)]],
                     rows[b].at[pl.ds(0, _HC)], gsems[2 * b])
    pltpu.async_copy(x_hbm.at[src_i.at[rrow, pl.ds(_HC, _HC)]],
                     rows[b].at[pl.ds(_HC, _HC)], gsems[2 * b + 1])


def _wait_gather(x_hbm, src_i, rows, gsems, b):
    pltpu.make_async_copy(x_hbm.at[src_i.at[0, pl.ds(0, _HC)]],
                          rows[b].at[pl.ds(0, _HC)], gsems[2 * b]).wait()
    pltpu.make_async_copy(x_hbm.at[src_i.at[0, pl.ds(0, _HC)]],
                          rows[b].at[pl.ds(_HC, _HC)], gsems[2 * b + 1]).wait()


def _seg_sum_body(x_hbm, src_hbm, dst_hbm, out_hbm,
                  acc_sh, src_i, dst_i, rows, gsems, ssems, isems):
    c = lax.axis_index("c")
    s = lax.axis_index("s")
    wid = s * _NC + c

    zeros16 = jnp.zeros((16,), jnp.float32)

    @pl.loop(0, _C)
    def _zero_rows(r):
        for cc in range(_H // 16):
            rows[0][r, pl.ds(cc * 16, 16)] = zeros16

    row0 = s * _RPS
    for j in range(_RPS // _C):
        pltpu.async_copy(rows[0], acc_sh.at[pl.ds(row0 + j * _C, _C)],
                         isems[0])

    base = wid * _PW

    pltpu.sync_copy(src_hbm.at[pl.ds(base, _IB)], src_i.at[pl.ds(0, _IB)])
    pltpu.sync_copy(dst_hbm.at[pl.ds(base, _IB)], dst_i.at[pl.ds(0, _IB)])
    for j in range(_RPS // _C):
        pltpu.make_async_copy(rows[0], acc_sh.at[pl.ds(row0, _C)],
                              isems[0]).wait()

    for b in range(_NBUF):
        _issue_gather(x_hbm, src_i, rows, gsems, b, b)
    plsc.subcore_barrier()

    @pl.loop(0, _NBLK)
    def _block(m):
        @pl.when(m < _NBLK - 1)
        def _issue_idx():
            slot = lax.rem(m + 1, 3) * _IB
            pltpu.async_copy(src_hbm.at[pl.ds(base + (m + 1) * _IB, _IB)],
                             src_i.at[pl.ds(slot, _IB)], isems[0])
            pltpu.async_copy(dst_hbm.at[pl.ds(base + (m + 1) * _IB, _IB)],
                             dst_i.at[pl.ds(slot, _IB)], isems[1])

        for i in range(0, _IB, _NBUF):
            if i == _IB - _NBUF:
                @pl.when(m < _NBLK - 1)
                def _wait_idx():
                    pltpu.make_async_copy(
                        src_hbm.at[pl.ds(base, _IB)],
                        src_i.at[pl.ds(0, _IB)], isems[0]).wait()
                    pltpu.make_async_copy(
                        dst_hbm.at[pl.ds(base, _IB)],
                        dst_i.at[pl.ds(0, _IB)], isems[1]).wait()
            for b in range(_NBUF):
                j = m * _IB + i + b
                _wait_gather(x_hbm, src_i, rows, gsems, b)
                pltpu.async_copy(rows[b],
                                 acc_sh.at[dst_i.at[lax.rem(j, _IR)]],
                                 ssems[b], add=True)
                pltpu.make_async_copy(rows[b], acc_sh.at[dst_i.at[0]],
                                      ssems[b]).wait()
                rn = lax.rem(lax.rem(j + _NBUF, _PW), _IR)
                _issue_gather(x_hbm, src_i, rows, gsems, b, rn)

    for b in range(_NBUF):
        _wait_gather(x_hbm, src_i, rows, gsems, b)

    plsc.subcore_barrier()
    pltpu.sync_copy(acc_sh.at[pl.ds(row0, _RPS)],
                    out_hbm.at[c, pl.ds(row0, _RPS)])


@functools.cache
def _make_seg_sum():
    return pl.kernel(
        _seg_sum_body,
        out_type=jax.ShapeDtypeStruct((_NC, _NP, _H), jnp.float32),
        mesh=plsc.VectorSubcoreMesh(core_axis_name="c", subcore_axis_name="s",
                                    num_cores=_NC, num_subcores=_NS),
        scratch_types=[
            pltpu.VMEM_SHARED((_NP, _H), jnp.float32),
            pltpu.VMEM((_IR, _C), jnp.int32),
            pltpu.VMEM((_IR, _C), jnp.int32),
            [pltpu.VMEM((_C, _H), jnp.float32) for _ in range(_NBUF)],
            [pltpu.SemaphoreType.DMA for _ in range(2 * _NBUF)],
            [pltpu.SemaphoreType.DMA for _ in range(_NBUF)],
            [pltpu.SemaphoreType.DMA for _ in range(2)],
        ],
    )


def _seg_sum(x, src2d, dst2d):
    return _make_seg_sum()(x, src2d, dst2d)


def _sigmoid(v):
    return 1.0 / (1.0 + jnp.exp(-v))


def _dot(a, w):
    return jnp.dot(a.astype(jnp.bfloat16), w,
                   preferred_element_type=jnp.float32)


def _gru_update(x, agg, eps, w1t, b1, w2t, b2, wrz, brz, w_in, b_in,
                w_hn, b_hn):
    pre = (1.0 + eps) * x + agg
    hid = jnp.maximum(_dot(pre, w1t) + b1, 0.0)
    m = jnp.maximum(_dot(hid, w2t) + b2, 0.0)
    mx = jnp.concatenate([m, x], axis=1)
    grz = _dot(mx, wrz) + brz
    r = _sigmoid(grz[:, :_H])
    z = _sigmoid(grz[:, _H:])
    n = jnp.tanh(_dot(m, w_in) + b_in + r * (_dot(x, w_hn) + b_hn))
    return (1.0 - z) * n + z * x


def _dense_step_body(eps_ref, x_ref, agg_ref, w1t_ref, b1_ref, w2t_ref,
                     b2_ref, wrz_ref, brz_ref, win_ref, bin_ref, whn_ref,
                     bhn_ref, out_ref):
    out_ref[...] = _gru_update(
        x_ref[...], agg_ref[0] + agg_ref[1], eps_ref[0],
        w1t_ref[...], b1_ref[...], w2t_ref[...], b2_ref[...],
        wrz_ref[...], brz_ref[...], win_ref[...], bin_ref[...],
        whn_ref[...], bhn_ref[...])


def _dense_final_body(eps_ref, x_ref, agg_ref, w1t_ref, b1_ref, w2t_ref,
                      b2_ref, wrz_ref, brz_ref, win_ref, bin_ref, whn_ref,
                      bhn_ref,
                      x1_ref, x2_ref, wc1t_ref, bc1_ref, wc2t_ref, bc2_ref,
                      x3_ref, ro_ref):
    x3 = _gru_update(
        x_ref[...], agg_ref[0] + agg_ref[1], eps_ref[0],
        w1t_ref[...], b1_ref[...], w2t_ref[...], b2_ref[...],
        wrz_ref[...], brz_ref[...], win_ref[...], bin_ref[...],
        whn_ref[...], bhn_ref[...])
    x3_ref[...] = x3
    cat = jnp.concatenate([x1_ref[...], x2_ref[...], x3], axis=1)
    hid = jnp.maximum(_dot(cat, wc1t_ref[...]) + bc1_ref[...], 0.0)
    ro_ref[...] = _dot(hid, wc2t_ref[...]) + bc2_ref[...]


def _full(shape):
    return pl.BlockSpec(shape, lambda i: (0,) * len(shape))


_row_spec = pl.BlockSpec((_BN, _H), lambda i: (i, 0))
_agg_spec = pl.BlockSpec((_NC, _BN, _H), lambda i: (0, i, 0))
_eps_spec = pl.BlockSpec(memory_space=pltpu.SMEM)

_common_specs = [
    _eps_spec, _row_spec, _agg_spec,
    _full((_H, _H)), _full((1, _H)), _full((_H, _H)), _full((1, _H)),
    _full((2 * _H, 2 * _H)), _full((1, 2 * _H)),
    _full((_H, _H)), _full((1, _H)),
    _full((_H, _H)), _full((1, _H)),
]

_dense_step = pl.pallas_call(
    _dense_step_body,
    grid=(_G,),
    in_specs=_common_specs,
    out_specs=_row_spec,
    out_shape=jax.ShapeDtypeStruct((_N, _H), jnp.float32),
)

_dense_final = pl.pallas_call(
    _dense_final_body,
    grid=(_G,),
    in_specs=_common_specs + [
        _row_spec, _row_spec,
        _full((3 * _H, 2 * _H)), _full((1, 2 * _H)),
        _full((2 * _H, _H)), _full((1, _H)),
    ],
    out_specs=[_row_spec, _row_spec],
    out_shape=[jax.ShapeDtypeStruct((_N, _H), jnp.float32),
               jax.ShapeDtypeStruct((_N, _H), jnp.float32)],
)


def kernel(x, edge_index, eps, W1, b1, W2, b2, W_ih, W_hh, b_ih, b_hh,
           Wc1, bc1, Wc2, bc2):
    src = edge_index[0]
    dst = edge_index[1]
    npad = _EP - _E
    pad_src = (jnp.arange(npad, dtype=jnp.int32) * 7) % _N
    pad_dst = _N + jnp.arange(npad, dtype=jnp.int32) % (_NP - _N)
    src2d = jnp.concatenate([src, pad_src]).reshape(_EP // _C, _C)
    dst2d = jnp.concatenate([dst, pad_dst]).reshape(_EP // _C, _C)
    eps1 = jnp.reshape(eps, (1,))
    bf = jnp.bfloat16
    w1t, w2t = W1.T.astype(bf), W2.T.astype(bf)
    wiht, whht = W_ih.T, W_hh.T
    wrz = jnp.concatenate([wiht[:, :2 * _H], whht[:, :2 * _H]],
                          axis=0).astype(bf)
    w_in = wiht[:, 2 * _H:].astype(bf)
    w_hn = whht[:, 2 * _H:].astype(bf)
    wc1t, wc2t = Wc1.T.astype(bf), Wc2.T.astype(bf)
    b1r = b1.reshape(1, _H)
    b2r = b2.reshape(1, _H)
    brz = (b_ih[:2 * _H] + b_hh[:2 * _H]).reshape(1, 2 * _H)
    binr = b_ih[2 * _H:].reshape(1, _H)
    bhnr = b_hh[2 * _H:].reshape(1, _H)
    bc1r = bc1.reshape(1, 2 * _H)
    bc2r = bc2.reshape(1, _H)

    common = (w1t, b1r, w2t, b2r, wrz, brz, w_in, binr, w_hn, bhnr)

    agg = _seg_sum(x, src2d, dst2d)
    x1 = _dense_step(eps1, x, agg, *common)
    agg = _seg_sum(x1, src2d, dst2d)
    x2 = _dense_step(eps1, x1, agg, *common)
    agg = _seg_sum(x2, src2d, dst2d)
    x3, readout = _dense_final(eps1, x2, agg, *common,
                               x1, x2, wc1t, bc1r, wc2t, bc2r)
    return (x3, readout)


# speedup vs baseline: 1.0399x; 1.0399x over previous
import functools

import jax
import jax.numpy as jnp
from jax import lax
from jax.experimental import pallas as pl
from jax.experimental.pallas import tpu as pltpu
from jax.experimental.pallas import tpu_sc as plsc

_N = 10000
_E = 320000
_H = 128
_ITER = 3

_NC = 2
_NS = 16
_NW = _NC * _NS
_C = 128
_PW = 80
_EP = _NW * _PW * _C
_NBUF = 2
_IB = 16
_IR = 3 * _IB
_NBLK = _PW // _IB
_NP = 10240
_RPS = _NP // _NS

_BN = 2000
_G = _N // _BN


def _issue_gather(x_hbm, src_i, rows, gsems, b, rrow):
    pltpu.async_copy(x_hbm.at[src_i.at[rrow]], rows[b], gsems[b])


def _wait_gather(x_hbm, src_i, rows, gsems, b):
    pltpu.make_async_copy(x_hbm.at[src_i.at[0]], rows[b], gsems[b]).wait()


def _seg_sum_body(x_hbm, src_hbm, dst_hbm, out_hbm,
                  acc_sh, src_i, dst_i, rows, gsems, ssems, isems):
    c = lax.axis_index("c")
    s = lax.axis_index("s")
    wid = s * _NC + c

    zeros16 = jnp.zeros((16,), jnp.float32)

    @pl.loop(0, _C)
    def _zero_rows(r):
        for cc in range(_H // 16):
            rows[0][r, pl.ds(cc * 16, 16)] = zeros16

    row0 = s * _RPS
    for j in range(_RPS // _C):
        pltpu.async_copy(rows[0], acc_sh.at[pl.ds(row0 + j * _C, _C)],
                         isems[0])

    base = wid * _PW

    pltpu.sync_copy(src_hbm.at[pl.ds(base, _IB)], src_i.at[pl.ds(0, _IB)])
    pltpu.sync_copy(dst_hbm.at[pl.ds(base, _IB)], dst_i.at[pl.ds(0, _IB)])
    for j in range(_RPS // _C):
        pltpu.make_async_copy(rows[0], acc_sh.at[pl.ds(row0, _C)],
                              isems[0]).wait()

    for b in range(_NBUF):
        _issue_gather(x_hbm, src_i, rows, gsems, b, b)
    plsc.subcore_barrier()

    @pl.loop(0, _NBLK)
    def _block(m):
        @pl.when(m < _NBLK - 1)
        def _issue_idx():
            slot = lax.rem(m + 1, 3) * _IB
            pltpu.async_copy(src_hbm.at[pl.ds(base + (m + 1) * _IB, _IB)],
                             src_i.at[pl.ds(slot, _IB)], isems[0])
            pltpu.async_copy(dst_hbm.at[pl.ds(base + (m + 1) * _IB, _IB)],
                             dst_i.at[pl.ds(slot, _IB)], isems[1])

        for i in range(0, _IB, _NBUF):
            if i == _IB - _NBUF:
                @pl.when(m < _NBLK - 1)
                def _wait_idx():
                    pltpu.make_async_copy(
                        src_hbm.at[pl.ds(base, _IB)],
                        src_i.at[pl.ds(0, _IB)], isems[0]).wait()
                    pltpu.make_async_copy(
                        dst_hbm.at[pl.ds(base, _IB)],
                        dst_i.at[pl.ds(0, _IB)], isems[1]).wait()
            for b in range(_NBUF):
                j = m * _IB + i + b
                _wait_gather(x_hbm, src_i, rows, gsems, b)
                pltpu.async_copy(rows[b],
                                 acc_sh.at[dst_i.at[lax.rem(j, _IR)]],
                                 ssems[b], add=True)
                pltpu.make_async_copy(rows[b], acc_sh.at[dst_i.at[0]],
                                      ssems[b]).wait()
                rn = lax.rem(lax.rem(j + _NBUF, _PW), _IR)
                _issue_gather(x_hbm, src_i, rows, gsems, b, rn)

    for b in range(_NBUF):
        _wait_gather(x_hbm, src_i, rows, gsems, b)

    plsc.subcore_barrier()
    pltpu.sync_copy(acc_sh.at[pl.ds(row0, _RPS)],
                    out_hbm.at[c, pl.ds(row0, _RPS)])


@functools.cache
def _make_seg_sum():
    return pl.kernel(
        _seg_sum_body,
        out_type=jax.ShapeDtypeStruct((_NC, _NP, _H), jnp.float32),
        mesh=plsc.VectorSubcoreMesh(core_axis_name="c", subcore_axis_name="s",
                                    num_cores=_NC, num_subcores=_NS),
        scratch_types=[
            pltpu.VMEM_SHARED((_NP, _H), jnp.float32),
            pltpu.VMEM((_IR, _C), jnp.int32),
            pltpu.VMEM((_IR, _C), jnp.int32),
            [pltpu.VMEM((_C, _H), jnp.float32) for _ in range(_NBUF)],
            [pltpu.SemaphoreType.DMA for _ in range(_NBUF)],
            [pltpu.SemaphoreType.DMA for _ in range(_NBUF)],
            [pltpu.SemaphoreType.DMA for _ in range(2)],
        ],
    )


def _seg_sum(x, src2d, dst2d):
    return _make_seg_sum()(x, src2d, dst2d)


def _sigmoid(v):
    return 1.0 / (1.0 + jnp.exp(-v))


def _dot(a, w):
    return jnp.dot(a.astype(jnp.bfloat16), w,
                   preferred_element_type=jnp.float32)


def _gru_update(x, agg, eps, w1t, b1, w2t, b2, wrz, brz, w_in, b_in,
                w_hn, b_hn):
    pre = (1.0 + eps) * x + agg
    hid = jnp.maximum(_dot(pre, w1t) + b1, 0.0)
    m = jnp.maximum(_dot(hid, w2t) + b2, 0.0)
    mx = jnp.concatenate([m, x], axis=1)
    grz = _dot(mx, wrz) + brz
    r = _sigmoid(grz[:, :_H])
    z = _sigmoid(grz[:, _H:])
    n = jnp.tanh(_dot(m, w_in) + b_in + r * (_dot(x, w_hn) + b_hn))
    return (1.0 - z) * n + z * x


def _dense_step_body(eps_ref, x_ref, agg_ref, w1t_ref, b1_ref, w2t_ref,
                     b2_ref, wrz_ref, brz_ref, win_ref, bin_ref, whn_ref,
                     bhn_ref, out_ref):
    out_ref[...] = _gru_update(
        x_ref[...], agg_ref[0] + agg_ref[1], eps_ref[0],
        w1t_ref[...], b1_ref[...], w2t_ref[...], b2_ref[...],
        wrz_ref[...], brz_ref[...], win_ref[...], bin_ref[...],
        whn_ref[...], bhn_ref[...])


def _dense_final_body(eps_ref, x_ref, agg_ref, w1t_ref, b1_ref, w2t_ref,
                      b2_ref, wrz_ref, brz_ref, win_ref, bin_ref, whn_ref,
                      bhn_ref,
                      x1_ref, x2_ref, wc1t_ref, bc1_ref, wc2t_ref, bc2_ref,
                      x3_ref, ro_ref):
    x3 = _gru_update(
        x_ref[...], agg_ref[0] + agg_ref[1], eps_ref[0],
        w1t_ref[...], b1_ref[...], w2t_ref[...], b2_ref[...],
        wrz_ref[...], brz_ref[...], win_ref[...], bin_ref[...],
        whn_ref[...], bhn_ref[...])
    x3_ref[...] = x3
    cat = jnp.concatenate([x1_ref[...], x2_ref[...], x3], axis=1)
    hid = jnp.maximum(_dot(cat, wc1t_ref[...]) + bc1_ref[...], 0.0)
    ro_ref[...] = _dot(hid, wc2t_ref[...]) + bc2_ref[...]


def _full(shape):
    return pl.BlockSpec(shape, lambda i: (0,) * len(shape))


_row_spec = pl.BlockSpec((_BN, _H), lambda i: (i, 0))
_agg_spec = pl.BlockSpec((_NC, _BN, _H), lambda i: (0, i, 0))
_eps_spec = pl.BlockSpec(memory_space=pltpu.SMEM)

_common_specs = [
    _eps_spec, _row_spec, _agg_spec,
    _full((_H, _H)), _full((1, _H)), _full((_H, _H)), _full((1, _H)),
    _full((2 * _H, 2 * _H)), _full((1, 2 * _H)),
    _full((_H, _H)), _full((1, _H)),
    _full((_H, _H)), _full((1, _H)),
]

_dense_step = pl.pallas_call(
    _dense_step_body,
    grid=(_G,),
    in_specs=_common_specs,
    out_specs=_row_spec,
    out_shape=jax.ShapeDtypeStruct((_N, _H), jnp.float32),
)

_dense_final = pl.pallas_call(
    _dense_final_body,
    grid=(_G,),
    in_specs=_common_specs + [
        _row_spec, _row_spec,
        _full((3 * _H, 2 * _H)), _full((1, 2 * _H)),
        _full((2 * _H, _H)), _full((1, _H)),
    ],
    out_specs=[_row_spec, _row_spec],
    out_shape=[jax.ShapeDtypeStruct((_N, _H), jnp.float32),
               jax.ShapeDtypeStruct((_N, _H), jnp.float32)],
)


def kernel(x, edge_index, eps, W1, b1, W2, b2, W_ih, W_hh, b_ih, b_hh,
           Wc1, bc1, Wc2, bc2):
    src = edge_index[0]
    dst = edge_index[1]
    npad = _EP - _E
    pad_src = (jnp.arange(npad, dtype=jnp.int32) * 7) % _N
    pad_dst = _N + jnp.arange(npad, dtype=jnp.int32) % (_NP - _N)
    src2d = jnp.concatenate([src, pad_src]).reshape(_EP // _C, _C)
    dst2d = jnp.concatenate([dst, pad_dst]).reshape(_EP // _C, _C)
    eps1 = jnp.reshape(eps, (1,))
    bf = jnp.bfloat16
    w1t, w2t = W1.T.astype(bf), W2.T.astype(bf)
    wiht, whht = W_ih.T, W_hh.T
    wrz = jnp.concatenate([wiht[:, :2 * _H], whht[:, :2 * _H]],
                          axis=0).astype(bf)
    w_in = wiht[:, 2 * _H:].astype(bf)
    w_hn = whht[:, 2 * _H:].astype(bf)
    wc1t, wc2t = Wc1.T.astype(bf), Wc2.T.astype(bf)
    b1r = b1.reshape(1, _H)
    b2r = b2.reshape(1, _H)
    brz = (b_ih[:2 * _H] + b_hh[:2 * _H]).reshape(1, 2 * _H)
    binr = b_ih[2 * _H:].reshape(1, _H)
    bhnr = b_hh[2 * _H:].reshape(1, _H)
    bc1r = bc1.reshape(1, 2 * _H)
    bc2r = bc2.reshape(1, _H)

    common = (w1t, b1r, w2t, b2r, wrz, brz, w_in, binr, w_hn, bhnr)

    agg = _seg_sum(x, src2d, dst2d)
    x1 = _dense_step(eps1, x, agg, *common)
    agg = _seg_sum(x1, src2d, dst2d)
    x2 = _dense_step(eps1, x1, agg, *common)
    agg = _seg_sum(x2, src2d, dst2d)
    x3, readout = _dense_final(eps1, x2, agg, *common,
                               x1, x2, wc1t, bc1r, wc2t, bc2r)
    return (x3, readout)
